# baseline (speedup 1.0000x reference)
import functools

import jax
import jax.numpy as jnp
from jax.experimental import pallas as pl
from jax.experimental.pallas import tpu as pltpu

T, D, E, K, F = 2048, 1024, 8, 2, 1024
BT = 256
NB = (K * T) // BT + (E - 1)
P = NB * BT


def _router_body(x_ref, wg_ref, topi_ref, topv_ref):
    logits = jnp.dot(x_ref[...], wg_ref[...], preferred_element_type=jnp.float32)
    e_idx = jax.lax.broadcasted_iota(jnp.int32, logits.shape, 1)
    m1 = jnp.max(logits, axis=-1, keepdims=True)
    i1 = jnp.min(jnp.where(logits == m1, e_idx, E), axis=-1, keepdims=True)
    rest = jnp.where(e_idx == i1, -jnp.inf, logits)
    m2 = jnp.max(rest, axis=-1, keepdims=True)
    i2 = jnp.min(jnp.where(rest == m2, e_idx, E), axis=-1, keepdims=True)
    w1 = 1.0 / (1.0 + jnp.exp(m2 - m1))
    topi_ref[...] = jnp.concatenate([i1, i2], axis=1)
    topv_ref[...] = jnp.concatenate([w1, 1.0 - w1], axis=1)


def _router(x, Wg):
    return pl.pallas_call(
        _router_body,
        out_shape=(
            jax.ShapeDtypeStruct((T, K), jnp.int32),
            jax.ShapeDtypeStruct((T, K), jnp.float32),
        ),
    )(x, Wg)


def _ffn_body(be_ref, xs_ref, wg_ref, wu_ref, wd_ref, ws_ref, out_ref):
    i = pl.program_id(0)

    @pl.when(i < be_ref[NB])
    def _():
        xb = xs_ref[...]
        wg = wg_ref[0].astype(jnp.bfloat16)
        wu = wu_ref[0].astype(jnp.bfloat16)
        wd = wd_ref[0].astype(jnp.bfloat16)
        g = jnp.dot(xb, wg, preferred_element_type=jnp.float32)
        u = jnp.dot(xb, wu, preferred_element_type=jnp.float32)
        h = ((g * jax.nn.sigmoid(g)) * u).astype(jnp.bfloat16)
        y = jnp.dot(h, wd, preferred_element_type=jnp.float32)
        out_ref[...] = y * ws_ref[...]


def _grouped_ffn(block_expert, xs, w_gate, w_up, w_down, ws):
    grid_spec = pltpu.PrefetchScalarGridSpec(
        num_scalar_prefetch=1,
        grid=(NB,),
        in_specs=[
            pl.BlockSpec((BT, D), lambda i, be: (i, 0)),
            pl.BlockSpec((1, D, F), lambda i, be: (be[i], 0, 0)),
            pl.BlockSpec((1, D, F), lambda i, be: (be[i], 0, 0)),
            pl.BlockSpec((1, F, D), lambda i, be: (be[i], 0, 0)),
            pl.BlockSpec((BT, 1), lambda i, be: (i, 0)),
        ],
        out_specs=pl.BlockSpec((BT, D), lambda i, be: (i, 0)),
    )
    return pl.pallas_call(
        _ffn_body,
        grid_spec=grid_spec,
        out_shape=jax.ShapeDtypeStruct((P, D), jnp.float32),
    )(block_expert, xs, w_gate, w_up, w_down, ws)


def kernel(x, Wg, w_gate, w_up, w_down):
    topi, topv = _router(x, Wg)

    ef = topi.reshape(-1)
    oh = (ef[:, None] == jnp.arange(E, dtype=jnp.int32)[None, :]).astype(jnp.int32)
    rank = jnp.sum((jnp.cumsum(oh, axis=0) - oh) * oh, axis=1)
    counts = jnp.sum(oh, axis=0)
    counts_pad = ((counts + BT - 1) // BT) * BT
    cum_pad = jnp.cumsum(counts_pad)
    seg_off = cum_pad - counts_pad
    pos = (seg_off[ef] + rank).astype(jnp.int32)
    tok = (jnp.arange(K * T, dtype=jnp.int32) // K)
    sort_tok = jnp.zeros((P,), jnp.int32).at[pos].set(tok)
    sort_w = jnp.zeros((P,), jnp.float32).at[pos].set(topv.reshape(-1))
    blk_start = jnp.arange(NB, dtype=jnp.int32) * BT
    nreal = (cum_pad[E - 1] // BT).astype(jnp.int32)
    raw_be = jnp.minimum(
        jnp.searchsorted(cum_pad, blk_start, side="right"), E - 1
    ).astype(jnp.int32)
    last_e = raw_be[nreal - 1]
    block_expert = jnp.concatenate([
        jnp.where(jnp.arange(NB, dtype=jnp.int32) < nreal, raw_be, last_e),
        nreal[None],
    ])

    xs = x.astype(jnp.bfloat16)[sort_tok]
    ys = _grouped_ffn(block_expert, xs, w_gate, w_up, w_down,
                      sort_w.reshape(P, 1))
    pos2 = pos.reshape(T, K)
    out = ys[pos2[:, 0]] + ys[pos2[:, 1]]
    return out


# speedup vs baseline: 1.2415x; 1.2415x over previous
import functools

import jax
import jax.numpy as jnp
from jax.experimental import pallas as pl
from jax.experimental.pallas import tpu as pltpu

T, D, E, K, F = 2048, 1024, 8, 2, 1024
BT = 256
NB = (K * T) // BT + (E - 1)
P = NB * BT


def _router_body(x_ref, wg_ref, topi_ref, topv_ref):
    logits = jnp.dot(x_ref[...], wg_ref[...], preferred_element_type=jnp.float32)
    e_idx = jax.lax.broadcasted_iota(jnp.int32, logits.shape, 1)
    m1 = jnp.max(logits, axis=-1, keepdims=True)
    i1 = jnp.min(jnp.where(logits == m1, e_idx, E), axis=-1, keepdims=True)
    rest = jnp.where(e_idx == i1, -jnp.inf, logits)
    m2 = jnp.max(rest, axis=-1, keepdims=True)
    i2 = jnp.min(jnp.where(rest == m2, e_idx, E), axis=-1, keepdims=True)
    w1 = 1.0 / (1.0 + jnp.exp(m2 - m1))
    topi_ref[...] = jnp.concatenate([i1, i2], axis=1)
    topv_ref[...] = jnp.concatenate([w1, 1.0 - w1], axis=1)


def _router(x, Wg):
    return pl.pallas_call(
        _router_body,
        out_shape=(
            jax.ShapeDtypeStruct((T, K), jnp.int32),
            jax.ShapeDtypeStruct((T, K), jnp.float32),
        ),
    )(x, Wg)


def _moe_body(be_ref, x_ref, st_ref, wg_ref, wu_ref, wd_ref, ws_ref, out_ref):
    i = pl.program_id(0)

    @pl.when(i == 0)
    def _():
        out_ref[...] = jnp.zeros_like(out_ref)

    @pl.when(i < be_ref[NB])
    def _():
        tok_iota = jax.lax.broadcasted_iota(jnp.int32, (BT, T), 1)
        onehot = (tok_iota == st_ref[0]).astype(jnp.bfloat16)
        xb = jnp.dot(onehot, x_ref[...],
                     preferred_element_type=jnp.float32
                     ).astype(jnp.bfloat16)
        wg = wg_ref[0].astype(jnp.bfloat16)
        wu = wu_ref[0].astype(jnp.bfloat16)
        wd = wd_ref[0].astype(jnp.bfloat16)
        g = jnp.dot(xb, wg, preferred_element_type=jnp.float32)
        u = jnp.dot(xb, wu, preferred_element_type=jnp.float32)
        h = ((g * jax.nn.sigmoid(g)) * u).astype(jnp.bfloat16)
        y = jnp.dot(h, wd, preferred_element_type=jnp.float32)
        ysc = (y * ws_ref[0]).astype(jnp.bfloat16)
        out_ref[...] += jax.lax.dot_general(
            onehot, ysc, (((0,), (0,)), ((), ())),
            preferred_element_type=jnp.float32)


def _moe_fused(block_expert, x16, sort_tok, w_gate, w_up, w_down, sort_w):
    grid_spec = pltpu.PrefetchScalarGridSpec(
        num_scalar_prefetch=1,
        grid=(NB,),
        in_specs=[
            pl.BlockSpec((T, D), lambda i, be: (0, 0)),
            pl.BlockSpec((1, BT, 1), lambda i, be: (i, 0, 0)),
            pl.BlockSpec((1, D, F), lambda i, be: (be[i], 0, 0)),
            pl.BlockSpec((1, D, F), lambda i, be: (be[i], 0, 0)),
            pl.BlockSpec((1, F, D), lambda i, be: (be[i], 0, 0)),
            pl.BlockSpec((1, BT, 1), lambda i, be: (i, 0, 0)),
        ],
        out_specs=pl.BlockSpec((T, D), lambda i, be: (0, 0)),
    )
    return pl.pallas_call(
        _moe_body,
        grid_spec=grid_spec,
        out_shape=jax.ShapeDtypeStruct((T, D), jnp.float32),
    )(block_expert, x16, sort_tok.reshape(NB, BT, 1), w_gate, w_up, w_down,
      sort_w.reshape(NB, BT, 1))


def kernel(x, Wg, w_gate, w_up, w_down):
    topi, topv = _router(x, Wg)

    ef = topi.reshape(-1)
    oh = (ef[:, None] == jnp.arange(E, dtype=jnp.int32)[None, :]).astype(jnp.int32)
    rank = jnp.sum((jnp.cumsum(oh, axis=0) - oh) * oh, axis=1)
    counts = jnp.sum(oh, axis=0)
    counts_pad = ((counts + BT - 1) // BT) * BT
    cum_pad = jnp.cumsum(counts_pad)
    seg_off = cum_pad - counts_pad
    pos = (seg_off[ef] + rank).astype(jnp.int32)
    tok = (jnp.arange(K * T, dtype=jnp.int32) // K)
    sort_tok = jnp.zeros((P,), jnp.int32).at[pos].set(tok)
    sort_w = jnp.zeros((P,), jnp.float32).at[pos].set(topv.reshape(-1))
    blk_start = jnp.arange(NB, dtype=jnp.int32) * BT
    nreal = (cum_pad[E - 1] // BT).astype(jnp.int32)
    raw_be = jnp.minimum(
        jnp.searchsorted(cum_pad, blk_start, side="right"), E - 1
    ).astype(jnp.int32)
    last_e = raw_be[nreal - 1]
    block_expert = jnp.concatenate([
        jnp.where(jnp.arange(NB, dtype=jnp.int32) < nreal, raw_be, last_e),
        nreal[None],
    ])

    return _moe_fused(block_expert, x.astype(jnp.bfloat16), sort_tok,
                      w_gate, w_up, w_down, sort_w)
